# Initial kernel scaffold; baseline (speedup 1.0000x reference)
import jax
import jax.numpy as jnp
from jax.experimental import pallas as pl


def kernel(x, ln1_w, ln1_b, ln2_w, ln2_b, q_a, q_b, kv_a, kv_b, out_w, out_b, router_w, w_fc, b_fc, w_proj, b_proj, moe_ln_w, moe_ln_b):
    raise NotImplementedError("write your pallas kernel here")


# speedup vs baseline: 2.4043x; 2.4043x over previous
import functools
import math

import jax
import jax.numpy as jnp
from jax.experimental import pallas as pl
from jax.experimental.pallas import tpu as pltpu

D = 768
H = 12
KVH = 4
HD = 64
E = 8
TOPK = 2
EPS = 1e-5
S = 2048
QR = 32
KR = 16
F = 4 * D

_NEG = -1e30


def _ln(x, w, b):
    m = jnp.mean(x, axis=-1, keepdims=True)
    v = jnp.mean((x - m) ** 2, axis=-1, keepdims=True)
    return (x - m) * jax.lax.rsqrt(v + EPS) * w + b


def _attn_kernel(x_ref, ln1w_ref, ln1b_ref, qa_ref, qb_ref, kva_ref,
                 kbk_ref, kbv_ref, o_ref, xn_scr, xa_scr, xk_scr):
    h = pl.program_id(0)

    @pl.when(h == 0)
    def _init():
        xn = _ln(x_ref[0], ln1w_ref[:], ln1b_ref[:])
        xn_scr[:] = xn
        xa_scr[:] = jnp.dot(xn, qa_ref[:], preferred_element_type=jnp.float32)
        xk_scr[:] = jnp.dot(xn, kva_ref[:], preferred_element_type=jnp.float32)

    q = jnp.dot(xa_scr[:], qb_ref[0], preferred_element_type=jnp.float32)
    k = jnp.dot(xk_scr[:], kbk_ref[0], preferred_element_type=jnp.float32)
    v = jnp.dot(xk_scr[:], kbv_ref[0], preferred_element_type=jnp.float32)

    d2 = HD // 2
    inv_freq = 1.0 / (10000.0 ** (
        jax.lax.broadcasted_iota(jnp.int32, (1, d2), 1).astype(jnp.float32)
        * (2.0 / HD)))
    t = jax.lax.broadcasted_iota(jnp.int32, (S, 1), 0).astype(jnp.float32)
    freqs = t * inv_freq
    cos = jnp.cos(freqs)
    sin = jnp.sin(freqs)

    q1, q2 = q[:, :d2], q[:, d2:]
    k1, k2 = k[:, :d2], k[:, d2:]
    qr = jnp.concatenate([q1 * cos - q2 * sin, q1 * sin + q2 * cos], axis=1)
    kr = jnp.concatenate([k1 * cos - k2 * sin, k1 * sin + k2 * cos], axis=1)

    scale = HD ** -0.5
    scores = jax.lax.dot_general(
        qr, kr, (((1,), (1,)), ((), ())),
        preferred_element_type=jnp.float32) * scale
    row = jax.lax.broadcasted_iota(jnp.int32, (S, S), 0)
    col = jax.lax.broadcasted_iota(jnp.int32, (S, S), 1)
    scores = jnp.where(col > row, _NEG, scores)
    m = jnp.max(scores, axis=-1, keepdims=True)
    p = jnp.exp(scores - m)
    w = p / jnp.sum(p, axis=-1, keepdims=True)
    o_ref[0] = jnp.dot(w, v, preferred_element_type=jnp.float32)


def _attn_call(x2, ln1_w, ln1_b, q_a, q_b_p, kv_a, kb_p, vb_f):
    return pl.pallas_call(
        _attn_kernel,
        grid=(H,),
        in_specs=[
            pl.BlockSpec((1, S, D), lambda h: (0, 0, 0)),
            pl.BlockSpec((D,), lambda h: (0,)),
            pl.BlockSpec((D,), lambda h: (0,)),
            pl.BlockSpec((D, QR), lambda h: (0, 0)),
            pl.BlockSpec((1, QR, HD), lambda h: (h, 0, 0)),
            pl.BlockSpec((D, KR), lambda h: (0, 0)),
            pl.BlockSpec((1, KR, HD), lambda h: (h // (H // KVH), 0, 0)),
            pl.BlockSpec((1, KR, HD), lambda h: (h // (H // KVH), 0, 0)),
        ],
        out_specs=pl.BlockSpec((1, S, HD), lambda h: (h, 0, 0)),
        out_shape=jax.ShapeDtypeStruct((H, S, HD), jnp.float32),
        scratch_shapes=[
            pltpu.VMEM((S, D), jnp.float32),
            pltpu.VMEM((S, QR), jnp.float32),
            pltpu.VMEM((S, KR), jnp.float32),
        ],
        compiler_params=pltpu.CompilerParams(
            dimension_semantics=("arbitrary",)),
    )(x2, ln1_w, ln1_b, q_a, q_b_p, kv_a, kb_p, vb_f)


def _proj_kernel(a_ref, ow_ref, ob_ref, x_ref, ln2w_ref, ln2b_ref, rw_ref,
                 h_ref, xn2_ref, lg_ref):
    hres = x_ref[0] + jnp.dot(a_ref[0], ow_ref[:],
                              preferred_element_type=jnp.float32) + ob_ref[:]
    h_ref[0] = hres
    xn2 = _ln(hres, ln2w_ref[:], ln2b_ref[:])
    xn2_ref[0] = xn2
    lg_ref[0] = jnp.dot(xn2, rw_ref[:], preferred_element_type=jnp.float32)


def _gelu(x):
    return 0.5 * x * (1.0 + jax.lax.erf(x * (2 ** -0.5)))


def _top2_weight(lg, e):
    i1 = jnp.argmax(lg, axis=-1)
    t1 = jnp.max(lg, axis=-1)
    colE = jax.lax.broadcasted_iota(jnp.int32, lg.shape, 1)
    lg2 = jnp.where(colE == i1[:, None], _NEG, lg)
    i2 = jnp.argmax(lg2, axis=-1)
    t2 = jnp.max(lg2, axis=-1)
    ex = jnp.exp(t2 - t1)
    p1 = 1.0 / (1.0 + ex)
    p2 = ex / (1.0 + ex)
    return (jnp.where(i1 == e, p1, 0.0) + jnp.where(i2 == e, p2, 0.0))


def _moe_kernel(xn2_ref, lg_ref, wfc_ref, bfc_ref, wpj_ref, bpj_ref,
                o_ref, acc_scr, *, ts):
    e = pl.program_id(0)
    t = pl.program_id(1)

    w = _top2_weight(lg_ref[0], e)

    h1 = jnp.dot(xn2_ref[0], wfc_ref[0],
                 preferred_element_type=jnp.float32) + bfc_ref[0]
    h1 = _gelu(h1)
    h2 = jnp.dot(h1, wpj_ref[0],
                 preferred_element_type=jnp.float32) + bpj_ref[0]
    contrib = h2 * w[:, None]

    @pl.when(e == 0)
    def _init():
        acc_scr[pl.ds(t * ts, ts), :] = contrib

    @pl.when(e > 0)
    def _acc():
        acc_scr[pl.ds(t * ts, ts), :] = acc_scr[pl.ds(t * ts, ts), :] + contrib

    @pl.when(e == E - 1)
    def _out():
        o_ref[0] = acc_scr[pl.ds(t * ts, ts), :]


def _comb_kernel(h_ref, moe_ref, lnw_ref, lnb_ref, o_ref):
    o_ref[0] = h_ref[0] + _ln(moe_ref[0], lnw_ref[:], lnb_ref[:])


def kernel(x, ln1_w, ln1_b, ln2_w, ln2_b, q_a, q_b, kv_a, kv_b, out_w, out_b,
           router_w, w_fc, b_fc, w_proj, b_proj, moe_ln_w, moe_ln_b):
    B = x.shape[0]
    perm = jnp.arange(HD).reshape(2, HD // 2).T.reshape(-1)
    q_b_p = q_b.reshape(QR, H, HD)[:, :, perm].transpose(1, 0, 2)
    kvb = kv_b.reshape(KR, KVH, HD, 2)
    kb = kvb[..., 0]
    vb = kvb[..., 1]
    kb_p = kb[:, :, perm].transpose(1, 0, 2)
    vb_f = vb.transpose(1, 0, 2)

    x2 = x.reshape(1, S, D)

    attn = _attn_call(x2, ln1_w, ln1_b, q_a, q_b_p, kv_a, kb_p, vb_f)
    attn = attn.transpose(1, 0, 2).reshape(1, S, D)

    hres, xn2, logits = pl.pallas_call(
        _proj_kernel,
        grid=(1,),
        in_specs=[
            pl.BlockSpec((1, S, D), lambda i: (0, 0, 0)),
            pl.BlockSpec((D, D), lambda i: (0, 0)),
            pl.BlockSpec((D,), lambda i: (0,)),
            pl.BlockSpec((1, S, D), lambda i: (0, 0, 0)),
            pl.BlockSpec((D,), lambda i: (0,)),
            pl.BlockSpec((D,), lambda i: (0,)),
            pl.BlockSpec((D, E), lambda i: (0, 0)),
        ],
        out_specs=[
            pl.BlockSpec((1, S, D), lambda i: (0, 0, 0)),
            pl.BlockSpec((1, S, D), lambda i: (0, 0, 0)),
            pl.BlockSpec((1, S, E), lambda i: (0, 0, 0)),
        ],
        out_shape=[
            jax.ShapeDtypeStruct((1, S, D), jnp.float32),
            jax.ShapeDtypeStruct((1, S, D), jnp.float32),
            jax.ShapeDtypeStruct((1, S, E), jnp.float32),
        ],
    )(attn, out_w, out_b, x2, ln2_w, ln2_b, router_w)

    TS = 512
    T = S // TS
    moe = pl.pallas_call(
        functools.partial(_moe_kernel, ts=TS),
        grid=(E, T),
        in_specs=[
            pl.BlockSpec((1, TS, D), lambda e, t: (0, t, 0)),
            pl.BlockSpec((1, TS, E), lambda e, t: (0, t, 0)),
            pl.BlockSpec((1, D, F), lambda e, t: (e, 0, 0)),
            pl.BlockSpec((1, 1, F), lambda e, t: (e, 0, 0)),
            pl.BlockSpec((1, F, D), lambda e, t: (e, 0, 0)),
            pl.BlockSpec((1, 1, D), lambda e, t: (e, 0, 0)),
        ],
        out_specs=pl.BlockSpec((1, TS, D), lambda e, t: (0, t, 0)),
        out_shape=jax.ShapeDtypeStruct((1, S, D), jnp.float32),
        scratch_shapes=[pltpu.VMEM((S, D), jnp.float32)],
        compiler_params=pltpu.CompilerParams(
            dimension_semantics=("arbitrary", "arbitrary")),
    )(xn2, logits, w_fc, b_fc.reshape(E, 1, F), w_proj,
      b_proj.reshape(E, 1, D))

    out = pl.pallas_call(
        _comb_kernel,
        grid=(1,),
        in_specs=[
            pl.BlockSpec((1, S, D), lambda i: (0, 0, 0)),
            pl.BlockSpec((1, S, D), lambda i: (0, 0, 0)),
            pl.BlockSpec((D,), lambda i: (0,)),
            pl.BlockSpec((D,), lambda i: (0,)),
        ],
        out_specs=pl.BlockSpec((1, S, D), lambda i: (0, 0, 0)),
        out_shape=jax.ShapeDtypeStruct((1, S, D), jnp.float32),
    )(hres, moe, moe_ln_w, moe_ln_b)

    return out.reshape(B, S, D), logits.reshape(B, S, E)


# speedup vs baseline: 3.1178x; 1.2967x over previous
import functools
import math

import jax
import jax.numpy as jnp
from jax.experimental import pallas as pl
from jax.experimental.pallas import tpu as pltpu

D = 768
H = 12
KVH = 4
HD = 64
E = 8
TOPK = 2
EPS = 1e-5
S = 2048
QR = 32
KR = 16
F = 4 * D

_NEG = -1e30


def _ln(x, w, b):
    m = jnp.mean(x, axis=-1, keepdims=True)
    v = jnp.mean((x - m) ** 2, axis=-1, keepdims=True)
    return (x - m) / jnp.sqrt(v + EPS) * w + b


def _attn_kernel(x_ref, ln1w_ref, ln1b_ref, qa_ref, qb_ref, kva_ref,
                 kbk_ref, kbv_ref, cos_ref, sin_ref, o_ref,
                 xn_scr, xa_scr, xk_scr):
    h = pl.program_id(0)

    @pl.when(h == 0)
    def _init():
        xn = _ln(x_ref[0], ln1w_ref[:], ln1b_ref[:])
        xn_scr[:] = xn
        xa_scr[:] = jnp.dot(xn, qa_ref[:], preferred_element_type=jnp.float32)
        xk_scr[:] = jnp.dot(xn, kva_ref[:], preferred_element_type=jnp.float32)

    q = jnp.dot(xa_scr[:], qb_ref[0], preferred_element_type=jnp.float32)
    k = jnp.dot(xk_scr[:], kbk_ref[0], preferred_element_type=jnp.float32)
    v = jnp.dot(xk_scr[:], kbv_ref[0], preferred_element_type=jnp.float32)

    d2 = HD // 2
    cos = cos_ref[:]
    sin = sin_ref[:]

    q1, q2 = q[:, :d2], q[:, d2:]
    k1, k2 = k[:, :d2], k[:, d2:]
    qr = jnp.concatenate([q1 * cos - q2 * sin, q1 * sin + q2 * cos], axis=1)
    kr = jnp.concatenate([k1 * cos - k2 * sin, k1 * sin + k2 * cos], axis=1)

    scale = HD ** -0.5
    scores = jax.lax.dot_general(
        qr, kr, (((1,), (1,)), ((), ())),
        preferred_element_type=jnp.float32) * scale
    row = jax.lax.broadcasted_iota(jnp.int32, (S, S), 0)
    col = jax.lax.broadcasted_iota(jnp.int32, (S, S), 1)
    scores = jnp.where(col > row, _NEG, scores)
    m = jnp.max(scores, axis=-1, keepdims=True)
    p = jnp.exp(scores - m)
    w = p / jnp.sum(p, axis=-1, keepdims=True)
    o_ref[0] = jnp.dot(w, v, preferred_element_type=jnp.float32)


def _attn_call(x2, ln1_w, ln1_b, q_a, q_b_p, kv_a, kb_p, vb_f, cos, sin):
    return pl.pallas_call(
        _attn_kernel,
        grid=(H,),
        in_specs=[
            pl.BlockSpec((1, S, D), lambda h: (0, 0, 0)),
            pl.BlockSpec((D,), lambda h: (0,)),
            pl.BlockSpec((D,), lambda h: (0,)),
            pl.BlockSpec((D, QR), lambda h: (0, 0)),
            pl.BlockSpec((1, QR, HD), lambda h: (h, 0, 0)),
            pl.BlockSpec((D, KR), lambda h: (0, 0)),
            pl.BlockSpec((1, KR, HD), lambda h: (h // (H // KVH), 0, 0)),
            pl.BlockSpec((1, KR, HD), lambda h: (h // (H // KVH), 0, 0)),
            pl.BlockSpec((S, HD // 2), lambda h: (0, 0)),
            pl.BlockSpec((S, HD // 2), lambda h: (0, 0)),
        ],
        out_specs=pl.BlockSpec((1, S, HD), lambda h: (h, 0, 0)),
        out_shape=jax.ShapeDtypeStruct((H, S, HD), jnp.float32),
        scratch_shapes=[
            pltpu.VMEM((S, D), jnp.float32),
            pltpu.VMEM((S, QR), jnp.float32),
            pltpu.VMEM((S, KR), jnp.float32),
        ],
        compiler_params=pltpu.CompilerParams(
            dimension_semantics=("arbitrary",)),
    )(x2, ln1_w, ln1_b, q_a, q_b_p, kv_a, kb_p, vb_f, cos, sin)


def _proj_kernel(a_ref, ow_ref, ob_ref, x_ref, ln2w_ref, ln2b_ref,
                 h_ref, xn2_ref):
    hres = x_ref[0] + jnp.dot(a_ref[0], ow_ref[:],
                              preferred_element_type=jnp.float32) + ob_ref[:]
    h_ref[0] = hres
    xn2 = _ln(hres, ln2w_ref[:], ln2b_ref[:])
    xn2_ref[0] = xn2


def _gelu(x):
    return 0.5 * x * (1.0 + jax.lax.erf(x * (2 ** -0.5)))


def _top2_weight(lg, e):
    i1 = jnp.argmax(lg, axis=-1)
    t1 = jnp.max(lg, axis=-1)
    colE = jax.lax.broadcasted_iota(jnp.int32, lg.shape, 1)
    lg2 = jnp.where(colE == i1[:, None], _NEG, lg)
    i2 = jnp.argmax(lg2, axis=-1)
    t2 = jnp.max(lg2, axis=-1)
    ex = jnp.exp(t2 - t1)
    p1 = 1.0 / (1.0 + ex)
    p2 = ex / (1.0 + ex)
    return (jnp.where(i1 == e, p1, 0.0) + jnp.where(i2 == e, p2, 0.0))


TILE = 256
NT = (2 * S) // TILE + E


def _cumsum_tokens(x):
    n = x.shape[0]
    k = 1
    while k < n:
        shifted = jnp.concatenate(
            [jnp.zeros((k, x.shape[1]), x.dtype), x[:-k]], axis=0)
        x = x + shifted
        k *= 2
    return x


def _route_kernel(lg_ref, ew_ref, rnk_ref, tm_ref):
    l = lg_ref[0]
    colE = jax.lax.broadcasted_iota(jnp.int32, (S, E), 1)
    i1 = jnp.argmax(l, axis=-1)
    t1 = jnp.max(l, axis=-1)
    l2 = jnp.where(colE == i1[:, None], _NEG, l)
    i2 = jnp.argmax(l2, axis=-1)
    t2 = jnp.max(l2, axis=-1)
    ex = jnp.exp(t2 - t1)
    p1 = (1.0 / (1.0 + ex))[:, None]
    p2 = (ex / (1.0 + ex))[:, None]
    m1 = colE == i1[:, None]
    m2 = colE == i2[:, None]
    match = m1 | m2
    matchf = match.astype(jnp.float32)
    ew = jnp.where(m1, p1, 0.0) + jnp.where(m2, p2, 0.0)
    rank = _cumsum_tokens(matchf) - matchf
    rnk = jnp.where(match, rank.astype(jnp.int32), -1)
    ew_ref[0] = ew.T
    rnk_ref[0] = rnk.T.astype(jnp.float32)

    counts = jnp.sum(matchf, axis=0)[None, :]
    ntiles = jnp.ceil(counts * (1.0 / TILE))
    cum = ntiles
    for k in (1, 2, 4):
        cum = cum + jnp.concatenate(
            [jnp.zeros((1, k), jnp.float32), cum[:, :-k]], axis=1)
    start = cum - ntiles
    j = jax.lax.broadcasted_iota(jnp.int32, (NT, 1), 0).astype(jnp.float32)
    started = (j >= start).astype(jnp.float32)
    te = jnp.minimum(jnp.sum(started, axis=1, keepdims=True) - 1.0,
                     float(E - 1))
    te = jnp.maximum(te, 0.0)
    colE2 = jax.lax.broadcasted_iota(jnp.int32, (NT, E), 1).astype(jnp.float32)
    onehot = (colE2 == te).astype(jnp.float32)
    start_at = jnp.sum(onehot * start, axis=1, keepdims=True)
    count_at = jnp.sum(onehot * counts, axis=1, keepdims=True)
    r0 = (j - start_at) * TILE
    active = (r0 < count_at).astype(jnp.float32)
    tm = jnp.concatenate([te, r0, active, jnp.zeros((NT, 1), jnp.float32)],
                         axis=1)
    tm_ref[0] = tm


def _gmoe_kernel(te_ref, r0_ref, act_ref, xn2_ref, rnk_ref, ew_ref,
                 wfc_ref, bfc_ref, wpj_ref, bpj_ref, o_ref, acc_scr):
    j = pl.program_id(0)

    @pl.when(j == 0)
    def _init():
        acc_scr[:] = jnp.zeros((S, D), jnp.float32)

    @pl.when(act_ref[j] == 1)
    def _compute():
        r0 = r0_ref[j]
        rrow = rnk_ref[0]
        ewrow = ew_ref[0]
        slot = jax.lax.broadcasted_iota(jnp.int32, (TILE, 1), 0).astype(
            jnp.float32) + r0.astype(jnp.float32)
        G = (slot == rrow).astype(jnp.float32)
        xs = jnp.dot(G, xn2_ref[0], preferred_element_type=jnp.float32)
        h1 = _gelu(jnp.dot(xs, wfc_ref[0],
                           preferred_element_type=jnp.float32) + bfc_ref[0])
        h2 = jnp.dot(h1, wpj_ref[0],
                     preferred_element_type=jnp.float32) + bpj_ref[0]
        Gw = G * ewrow
        acc_scr[:] = acc_scr[:] + jax.lax.dot_general(
            Gw, h2, (((0,), (0,)), ((), ())),
            preferred_element_type=jnp.float32)

    @pl.when(j == NT - 1)
    def _out():
        o_ref[0] = acc_scr[:]


def _gmoe_call(xn2, logits, w_fc, b_fc, w_proj, b_proj):
    ew, rnk, tm = pl.pallas_call(
        _route_kernel,
        grid=(1,),
        in_specs=[pl.BlockSpec((1, S, E), lambda i: (0, 0, 0))],
        out_specs=[
            pl.BlockSpec((1, E, S), lambda i: (0, 0, 0)),
            pl.BlockSpec((1, E, S), lambda i: (0, 0, 0)),
            pl.BlockSpec((1, NT, 4), lambda i: (0, 0, 0)),
        ],
        out_shape=[
            jax.ShapeDtypeStruct((1, E, S), jnp.float32),
            jax.ShapeDtypeStruct((1, E, S), jnp.float32),
            jax.ShapeDtypeStruct((1, NT, 4), jnp.float32),
        ],
    )(logits)

    tm_i = tm.reshape(NT, 4).astype(jnp.int32)
    te = tm_i[:, 0]
    r0 = tm_i[:, 1]
    act = tm_i[:, 2]

    grid_spec = pltpu.PrefetchScalarGridSpec(
        num_scalar_prefetch=3,
        grid=(NT,),
        in_specs=[
            pl.BlockSpec((1, S, D), lambda j, te, r0, act: (0, 0, 0)),
            pl.BlockSpec((1, 1, S), lambda j, te, r0, act: (te[j], 0, 0)),
            pl.BlockSpec((1, 1, S), lambda j, te, r0, act: (te[j], 0, 0)),
            pl.BlockSpec((1, D, F), lambda j, te, r0, act: (te[j], 0, 0)),
            pl.BlockSpec((1, 1, F), lambda j, te, r0, act: (te[j], 0, 0)),
            pl.BlockSpec((1, F, D), lambda j, te, r0, act: (te[j], 0, 0)),
            pl.BlockSpec((1, 1, D), lambda j, te, r0, act: (te[j], 0, 0)),
        ],
        out_specs=pl.BlockSpec((1, S, D), lambda j, te, r0, act: (0, 0, 0)),
        scratch_shapes=[pltpu.VMEM((S, D), jnp.float32)],
    )
    moe = pl.pallas_call(
        _gmoe_kernel,
        grid_spec=grid_spec,
        out_shape=jax.ShapeDtypeStruct((1, S, D), jnp.float32),
        compiler_params=pltpu.CompilerParams(
            dimension_semantics=("arbitrary",)),
    )(te, r0, act, xn2, rnk.reshape(E, 1, S), ew.reshape(E, 1, S), w_fc,
      b_fc.reshape(E, 1, F), w_proj, b_proj.reshape(E, 1, D))
    return moe


def _moe_kernel(xn2_ref, lg_ref, wfc_ref, bfc_ref, wpj_ref, bpj_ref,
                o_ref, acc_scr, *, ts):
    e = pl.program_id(0)
    t = pl.program_id(1)

    w = _top2_weight(lg_ref[0], e)

    h1 = jnp.dot(xn2_ref[0], wfc_ref[0],
                 preferred_element_type=jnp.float32) + bfc_ref[0]
    h1 = _gelu(h1)
    h2 = jnp.dot(h1, wpj_ref[0],
                 preferred_element_type=jnp.float32) + bpj_ref[0]
    contrib = h2 * w[:, None]

    @pl.when(e == 0)
    def _init():
        acc_scr[pl.ds(t * ts, ts), :] = contrib

    @pl.when(e > 0)
    def _acc():
        acc_scr[pl.ds(t * ts, ts), :] = acc_scr[pl.ds(t * ts, ts), :] + contrib

    @pl.when(e == E - 1)
    def _out():
        o_ref[0] = acc_scr[pl.ds(t * ts, ts), :]


def _comb_kernel(h_ref, moe_ref, lnw_ref, lnb_ref, o_ref):
    o_ref[0] = h_ref[0] + _ln(moe_ref[0], lnw_ref[:], lnb_ref[:])


def kernel(x, ln1_w, ln1_b, ln2_w, ln2_b, q_a, q_b, kv_a, kv_b, out_w, out_b,
           router_w, w_fc, b_fc, w_proj, b_proj, moe_ln_w, moe_ln_b):
    B = x.shape[0]
    perm = jnp.arange(HD).reshape(2, HD // 2).T.reshape(-1)
    q_b_p = q_b.reshape(QR, H, HD)[:, :, perm].transpose(1, 0, 2)
    kvb = kv_b.reshape(KR, KVH, HD, 2)
    kb = kvb[..., 0]
    vb = kvb[..., 1]
    kb_p = kb[:, :, perm].transpose(1, 0, 2)
    vb_f = vb.transpose(1, 0, 2)

    x2 = x.reshape(1, S, D)

    inv_freq = 1.0 / (10000.0 ** (
        jnp.arange(0, HD, 2, dtype=jnp.float32) / HD))
    t = jnp.arange(S, dtype=jnp.float32)
    freqs = jnp.outer(t, inv_freq)
    cos = jnp.cos(freqs)
    sin = jnp.sin(freqs)

    attn = _attn_call(x2, ln1_w, ln1_b, q_a, q_b_p, kv_a, kb_p, vb_f,
                      cos, sin)
    attn = attn.transpose(1, 0, 2).reshape(1, S, D)

    hres, xn2 = pl.pallas_call(
        _proj_kernel,
        grid=(1,),
        in_specs=[
            pl.BlockSpec((1, S, D), lambda i: (0, 0, 0)),
            pl.BlockSpec((D, D), lambda i: (0, 0)),
            pl.BlockSpec((D,), lambda i: (0,)),
            pl.BlockSpec((1, S, D), lambda i: (0, 0, 0)),
            pl.BlockSpec((D,), lambda i: (0,)),
            pl.BlockSpec((D,), lambda i: (0,)),
        ],
        out_specs=[
            pl.BlockSpec((1, S, D), lambda i: (0, 0, 0)),
            pl.BlockSpec((1, S, D), lambda i: (0, 0, 0)),
        ],
        out_shape=[
            jax.ShapeDtypeStruct((1, S, D), jnp.float32),
            jax.ShapeDtypeStruct((1, S, D), jnp.float32),
        ],
    )(attn, out_w, out_b, x2, ln2_w, ln2_b)

    logits = xn2 @ router_w

    moe = _gmoe_call(xn2, logits, w_fc, b_fc, w_proj, b_proj)

    out = pl.pallas_call(
        _comb_kernel,
        grid=(1,),
        in_specs=[
            pl.BlockSpec((1, S, D), lambda i: (0, 0, 0)),
            pl.BlockSpec((1, S, D), lambda i: (0, 0, 0)),
            pl.BlockSpec((D,), lambda i: (0,)),
            pl.BlockSpec((D,), lambda i: (0,)),
        ],
        out_specs=pl.BlockSpec((1, S, D), lambda i: (0, 0, 0)),
        out_shape=jax.ShapeDtypeStruct((1, S, D), jnp.float32),
    )(hres, moe, moe_ln_w, moe_ln_b)

    return out.reshape(B, S, D), logits.reshape(B, S, E)


# speedup vs baseline: 3.1239x; 1.0020x over previous
import functools
import math

import jax
import jax.numpy as jnp
from jax.experimental import pallas as pl
from jax.experimental.pallas import tpu as pltpu

D = 768
H = 12
KVH = 4
HD = 64
E = 8
TOPK = 2
EPS = 1e-5
S = 2048
QR = 32
KR = 16
F = 4 * D

_NEG = -1e30


def _ln(x, w, b):
    m = jnp.mean(x, axis=-1, keepdims=True)
    v = jnp.mean((x - m) ** 2, axis=-1, keepdims=True)
    return (x - m) / jnp.sqrt(v + EPS) * w + b


def _attn_kernel(x_ref, ln1w_ref, ln1b_ref, qa_ref, qb_ref, kva_ref,
                 kbk_ref, kbv_ref, cos_ref, sin_ref, o_ref,
                 xn_scr, xa_scr, xk_scr):
    h = pl.program_id(0)

    @pl.when(h == 0)
    def _init():
        xn = _ln(x_ref[0], ln1w_ref[:], ln1b_ref[:])
        xn_scr[:] = xn
        xa_scr[:] = jnp.dot(xn, qa_ref[:], preferred_element_type=jnp.float32)
        xk_scr[:] = jnp.dot(xn, kva_ref[:], preferred_element_type=jnp.float32)

    q = jnp.dot(xa_scr[:], qb_ref[0], preferred_element_type=jnp.float32)
    k = jnp.dot(xk_scr[:], kbk_ref[0], preferred_element_type=jnp.float32)
    v = jnp.dot(xk_scr[:], kbv_ref[0], preferred_element_type=jnp.float32)

    cos2 = cos_ref[:]
    sina = sin_ref[:]
    lane = jax.lax.broadcasted_iota(jnp.int32, (S, HD), 1)
    even = (lane % 2) == 0

    def _rope_i(z):
        z_sw = jnp.where(even, jnp.roll(z, -1, axis=1), jnp.roll(z, 1, axis=1))
        return z * cos2 + z_sw * sina

    qr = _rope_i(q)
    kr = _rope_i(k)

    scale = HD ** -0.5
    scores = jax.lax.dot_general(
        qr, kr, (((1,), (1,)), ((), ())),
        preferred_element_type=jnp.float32) * scale
    row = jax.lax.broadcasted_iota(jnp.int32, (S, S), 0)
    col = jax.lax.broadcasted_iota(jnp.int32, (S, S), 1)
    scores = jnp.where(col > row, _NEG, scores)
    m = jnp.max(scores, axis=-1, keepdims=True)
    p = jnp.exp(scores - m)
    w = p / jnp.sum(p, axis=-1, keepdims=True)
    o_ref[0] = jnp.dot(w, v, preferred_element_type=jnp.float32)


def _attn_call(x2, ln1_w, ln1_b, q_a, q_b_p, kv_a, kb_p, vb_f, cos, sin):
    return pl.pallas_call(
        _attn_kernel,
        grid=(H,),
        in_specs=[
            pl.BlockSpec((1, S, D), lambda h: (0, 0, 0)),
            pl.BlockSpec((D,), lambda h: (0,)),
            pl.BlockSpec((D,), lambda h: (0,)),
            pl.BlockSpec((D, QR), lambda h: (0, 0)),
            pl.BlockSpec((1, QR, HD), lambda h: (h, 0, 0)),
            pl.BlockSpec((D, KR), lambda h: (0, 0)),
            pl.BlockSpec((1, KR, HD), lambda h: (h // (H // KVH), 0, 0)),
            pl.BlockSpec((1, KR, HD), lambda h: (h // (H // KVH), 0, 0)),
            pl.BlockSpec((S, HD), lambda h: (0, 0)),
            pl.BlockSpec((S, HD), lambda h: (0, 0)),
        ],
        out_specs=pl.BlockSpec((1, S, HD), lambda h: (h, 0, 0)),
        out_shape=jax.ShapeDtypeStruct((H, S, HD), jnp.float32),
        scratch_shapes=[
            pltpu.VMEM((S, D), jnp.float32),
            pltpu.VMEM((S, QR), jnp.float32),
            pltpu.VMEM((S, KR), jnp.float32),
        ],
        compiler_params=pltpu.CompilerParams(
            dimension_semantics=("arbitrary",)),
    )(x2, ln1_w, ln1_b, q_a, q_b_p, kv_a, kb_p, vb_f, cos, sin)


def _proj_kernel(a_ref, ow_ref, ob_ref, x_ref, ln2w_ref, ln2b_ref,
                 h_ref, xn2_ref):
    hres = x_ref[0] + jnp.dot(a_ref[0], ow_ref[:],
                              preferred_element_type=jnp.float32) + ob_ref[:]
    h_ref[0] = hres
    xn2 = _ln(hres, ln2w_ref[:], ln2b_ref[:])
    xn2_ref[0] = xn2


def _gelu(x):
    return 0.5 * x * (1.0 + jax.lax.erf(x * (2 ** -0.5)))


def _top2_weight(lg, e):
    i1 = jnp.argmax(lg, axis=-1)
    t1 = jnp.max(lg, axis=-1)
    colE = jax.lax.broadcasted_iota(jnp.int32, lg.shape, 1)
    lg2 = jnp.where(colE == i1[:, None], _NEG, lg)
    i2 = jnp.argmax(lg2, axis=-1)
    t2 = jnp.max(lg2, axis=-1)
    ex = jnp.exp(t2 - t1)
    p1 = 1.0 / (1.0 + ex)
    p2 = ex / (1.0 + ex)
    return (jnp.where(i1 == e, p1, 0.0) + jnp.where(i2 == e, p2, 0.0))


TILE = 256
NT = (2 * S) // TILE + E


def _cumsum_tokens(x):
    n = x.shape[0]
    k = 1
    while k < n:
        shifted = jnp.concatenate(
            [jnp.zeros((k, x.shape[1]), x.dtype), x[:-k]], axis=0)
        x = x + shifted
        k *= 2
    return x


def _route_kernel(lg_ref, ew_ref, rnk_ref, tm_ref):
    l = lg_ref[0]
    colE = jax.lax.broadcasted_iota(jnp.int32, (S, E), 1)
    i1 = jnp.argmax(l, axis=-1)
    t1 = jnp.max(l, axis=-1)
    l2 = jnp.where(colE == i1[:, None], _NEG, l)
    i2 = jnp.argmax(l2, axis=-1)
    t2 = jnp.max(l2, axis=-1)
    ex = jnp.exp(t2 - t1)
    p1 = (1.0 / (1.0 + ex))[:, None]
    p2 = (ex / (1.0 + ex))[:, None]
    m1 = colE == i1[:, None]
    m2 = colE == i2[:, None]
    match = m1 | m2
    matchf = match.astype(jnp.float32)
    ew = jnp.where(m1, p1, 0.0) + jnp.where(m2, p2, 0.0)
    rank = _cumsum_tokens(matchf) - matchf
    rnk = jnp.where(match, rank.astype(jnp.int32), -1)
    ew_ref[0] = ew.T
    rnk_ref[0] = rnk.T.astype(jnp.float32)

    counts = jnp.sum(matchf, axis=0)[None, :]
    ntiles = jnp.ceil(counts * (1.0 / TILE))
    cum = ntiles
    for k in (1, 2, 4):
        cum = cum + jnp.concatenate(
            [jnp.zeros((1, k), jnp.float32), cum[:, :-k]], axis=1)
    start = cum - ntiles
    j = jax.lax.broadcasted_iota(jnp.int32, (NT, 1), 0).astype(jnp.float32)
    started = (j >= start).astype(jnp.float32)
    te = jnp.minimum(jnp.sum(started, axis=1, keepdims=True) - 1.0,
                     float(E - 1))
    te = jnp.maximum(te, 0.0)
    colE2 = jax.lax.broadcasted_iota(jnp.int32, (NT, E), 1).astype(jnp.float32)
    onehot = (colE2 == te).astype(jnp.float32)
    start_at = jnp.sum(onehot * start, axis=1, keepdims=True)
    count_at = jnp.sum(onehot * counts, axis=1, keepdims=True)
    r0 = (j - start_at) * TILE
    active = (r0 < count_at).astype(jnp.float32)
    tm = jnp.concatenate([te, r0, active, jnp.zeros((NT, 1), jnp.float32)],
                         axis=1)
    tm_ref[0] = tm


def _gmoe_kernel(te_ref, r0_ref, act_ref, xn2_ref, rnk_ref, ew_ref,
                 wfc_ref, bfc_ref, wpj_ref, bpj_ref, o_ref, acc_scr):
    j = pl.program_id(0)

    @pl.when(j == 0)
    def _init():
        acc_scr[:] = jnp.zeros((S, D), jnp.float32)

    @pl.when(act_ref[j] == 1)
    def _compute():
        r0 = r0_ref[j]
        rrow = rnk_ref[0]
        ewrow = ew_ref[0]
        slot = jax.lax.broadcasted_iota(jnp.int32, (TILE, 1), 0).astype(
            jnp.float32) + r0.astype(jnp.float32)
        G = (slot == rrow).astype(jnp.float32)
        xs = jnp.dot(G, xn2_ref[0], preferred_element_type=jnp.float32)
        h1 = _gelu(jnp.dot(xs, wfc_ref[0],
                           preferred_element_type=jnp.float32) + bfc_ref[0])
        h2 = jnp.dot(h1, wpj_ref[0],
                     preferred_element_type=jnp.float32) + bpj_ref[0]
        Gw = G * ewrow
        acc_scr[:] = acc_scr[:] + jax.lax.dot_general(
            Gw, h2, (((0,), (0,)), ((), ())),
            preferred_element_type=jnp.float32)

    @pl.when(j == NT - 1)
    def _out():
        o_ref[0] = acc_scr[:]


def _gmoe_call(xn2, logits, w_fc, b_fc, w_proj, b_proj):
    ew, rnk, tm = pl.pallas_call(
        _route_kernel,
        grid=(1,),
        in_specs=[pl.BlockSpec((1, S, E), lambda i: (0, 0, 0))],
        out_specs=[
            pl.BlockSpec((1, E, S), lambda i: (0, 0, 0)),
            pl.BlockSpec((1, E, S), lambda i: (0, 0, 0)),
            pl.BlockSpec((1, NT, 4), lambda i: (0, 0, 0)),
        ],
        out_shape=[
            jax.ShapeDtypeStruct((1, E, S), jnp.float32),
            jax.ShapeDtypeStruct((1, E, S), jnp.float32),
            jax.ShapeDtypeStruct((1, NT, 4), jnp.float32),
        ],
    )(logits)

    tm_i = tm.reshape(NT, 4).astype(jnp.int32)
    te = tm_i[:, 0]
    r0 = tm_i[:, 1]
    act = tm_i[:, 2]

    grid_spec = pltpu.PrefetchScalarGridSpec(
        num_scalar_prefetch=3,
        grid=(NT,),
        in_specs=[
            pl.BlockSpec((1, S, D), lambda j, te, r0, act: (0, 0, 0)),
            pl.BlockSpec((1, 1, S), lambda j, te, r0, act: (te[j], 0, 0)),
            pl.BlockSpec((1, 1, S), lambda j, te, r0, act: (te[j], 0, 0)),
            pl.BlockSpec((1, D, F), lambda j, te, r0, act: (te[j], 0, 0)),
            pl.BlockSpec((1, 1, F), lambda j, te, r0, act: (te[j], 0, 0)),
            pl.BlockSpec((1, F, D), lambda j, te, r0, act: (te[j], 0, 0)),
            pl.BlockSpec((1, 1, D), lambda j, te, r0, act: (te[j], 0, 0)),
        ],
        out_specs=pl.BlockSpec((1, S, D), lambda j, te, r0, act: (0, 0, 0)),
        scratch_shapes=[pltpu.VMEM((S, D), jnp.float32)],
    )
    moe = pl.pallas_call(
        _gmoe_kernel,
        grid_spec=grid_spec,
        out_shape=jax.ShapeDtypeStruct((1, S, D), jnp.float32),
        compiler_params=pltpu.CompilerParams(
            dimension_semantics=("arbitrary",)),
    )(te, r0, act, xn2, rnk.reshape(E, 1, S), ew.reshape(E, 1, S), w_fc,
      b_fc.reshape(E, 1, F), w_proj, b_proj.reshape(E, 1, D))
    return moe


def _moe_kernel(xn2_ref, lg_ref, wfc_ref, bfc_ref, wpj_ref, bpj_ref,
                o_ref, acc_scr, *, ts):
    e = pl.program_id(0)
    t = pl.program_id(1)

    w = _top2_weight(lg_ref[0], e)

    h1 = jnp.dot(xn2_ref[0], wfc_ref[0],
                 preferred_element_type=jnp.float32) + bfc_ref[0]
    h1 = _gelu(h1)
    h2 = jnp.dot(h1, wpj_ref[0],
                 preferred_element_type=jnp.float32) + bpj_ref[0]
    contrib = h2 * w[:, None]

    @pl.when(e == 0)
    def _init():
        acc_scr[pl.ds(t * ts, ts), :] = contrib

    @pl.when(e > 0)
    def _acc():
        acc_scr[pl.ds(t * ts, ts), :] = acc_scr[pl.ds(t * ts, ts), :] + contrib

    @pl.when(e == E - 1)
    def _out():
        o_ref[0] = acc_scr[pl.ds(t * ts, ts), :]


def _comb_kernel(h_ref, moe_ref, lnw_ref, lnb_ref, o_ref):
    o_ref[0] = h_ref[0] + _ln(moe_ref[0], lnw_ref[:], lnb_ref[:])


def kernel(x, ln1_w, ln1_b, ln2_w, ln2_b, q_a, q_b, kv_a, kv_b, out_w, out_b,
           router_w, w_fc, b_fc, w_proj, b_proj, moe_ln_w, moe_ln_b):
    B = x.shape[0]
    q_b_p = q_b.reshape(QR, H, HD).transpose(1, 0, 2)
    kvb = kv_b.reshape(KR, KVH, HD, 2)
    kb_p = kvb[..., 0].transpose(1, 0, 2)
    vb_f = kvb[..., 1].transpose(1, 0, 2)

    x2 = x.reshape(1, S, D)

    inv_freq = 1.0 / (10000.0 ** (
        jnp.arange(0, HD, 2, dtype=jnp.float32) / HD))
    t = jnp.arange(S, dtype=jnp.float32)
    freqs = jnp.outer(t, inv_freq)
    cos = jnp.cos(freqs)
    sin = jnp.sin(freqs)
    cos2 = jnp.repeat(cos, 2, axis=1)
    sina = jnp.stack([-sin, sin], axis=-1).reshape(S, HD)

    attn = _attn_call(x2, ln1_w, ln1_b, q_a, q_b_p, kv_a, kb_p, vb_f,
                      cos2, sina)
    attn = attn.transpose(1, 0, 2).reshape(1, S, D)

    hres, xn2 = pl.pallas_call(
        _proj_kernel,
        grid=(1,),
        in_specs=[
            pl.BlockSpec((1, S, D), lambda i: (0, 0, 0)),
            pl.BlockSpec((D, D), lambda i: (0, 0)),
            pl.BlockSpec((D,), lambda i: (0,)),
            pl.BlockSpec((1, S, D), lambda i: (0, 0, 0)),
            pl.BlockSpec((D,), lambda i: (0,)),
            pl.BlockSpec((D,), lambda i: (0,)),
        ],
        out_specs=[
            pl.BlockSpec((1, S, D), lambda i: (0, 0, 0)),
            pl.BlockSpec((1, S, D), lambda i: (0, 0, 0)),
        ],
        out_shape=[
            jax.ShapeDtypeStruct((1, S, D), jnp.float32),
            jax.ShapeDtypeStruct((1, S, D), jnp.float32),
        ],
    )(attn, out_w, out_b, x2, ln2_w, ln2_b)

    logits = xn2 @ router_w

    moe = _gmoe_call(xn2, logits, w_fc, b_fc, w_proj, b_proj)

    out = pl.pallas_call(
        _comb_kernel,
        grid=(1,),
        in_specs=[
            pl.BlockSpec((1, S, D), lambda i: (0, 0, 0)),
            pl.BlockSpec((1, S, D), lambda i: (0, 0, 0)),
            pl.BlockSpec((D,), lambda i: (0,)),
            pl.BlockSpec((D,), lambda i: (0,)),
        ],
        out_specs=pl.BlockSpec((1, S, D), lambda i: (0, 0, 0)),
        out_shape=jax.ShapeDtypeStruct((1, S, D), jnp.float32),
    )(hres, moe, moe_ln_w, moe_ln_b)

    return out.reshape(B, S, D), logits.reshape(B, S, E)
